# Initial kernel scaffold; baseline (speedup 1.0000x reference)
import jax
import jax.numpy as jnp
from jax.experimental import pallas as pl


def kernel(histories, table0, table1):
    raise NotImplementedError("write your pallas kernel here")


# speedup vs baseline: 1.5672x; 1.5672x over previous
import functools

import jax
import jax.numpy as jnp
from jax import lax
from jax.experimental import pallas as pl
from jax.experimental.pallas import tpu as pltpu
from jax.experimental.pallas import tpu_sc as plsc

BATCH = 4096
SEQ = 200
EMB = 32
NC = 2
NS = 16
NW = NC * NS
ROWS = BATCH * SEQ
RPW = ROWS // NW
CHUNK = 128
NCH = RPW // CHUNK


def _sc_gather_body(table0, table1, idx0, idx1, out, idx0_v, idx1_v, buf, sem):
    c = lax.axis_index("c")
    s = lax.axis_index("s")
    wid = s * NC + c
    base = wid * RPW
    pltpu.sync_copy(idx0.at[wid], idx0_v)
    pltpu.sync_copy(idx1.at[wid], idx1_v)

    def make_chunk(field, idx_v, table):
        def chunk(j, carry):
            pltpu.async_copy(table.at[idx_v.at[j]], buf, sem).wait()
            pltpu.sync_copy(buf, out.at[pl.ds(base + j * CHUNK, CHUNK), field])
            return carry
        return chunk

    lax.fori_loop(0, NCH, make_chunk(0, idx0_v, table0), 0)
    lax.fori_loop(0, NCH, make_chunk(1, idx1_v, table1), 0)


_sc_gather = pl.kernel(
    _sc_gather_body,
    out_type=jax.ShapeDtypeStruct((ROWS, 2, EMB), jnp.float32),
    mesh=plsc.VectorSubcoreMesh(core_axis_name="c", subcore_axis_name="s"),
    scratch_types=[
        pltpu.VMEM((NCH, CHUNK), jnp.int32),
        pltpu.VMEM((NCH, CHUNK), jnp.int32),
        pltpu.VMEM((CHUNK, EMB), jnp.float32),
        pltpu.SemaphoreType.DMA,
    ],
    compiler_params=pltpu.CompilerParams(use_tc_tiling_on_sc=False),
)


def _mask_body(h_ref, m_ref):
    m_ref[...] = h_ref[...] != 0


_mask_call = pl.pallas_call(
    _mask_body,
    out_shape=jax.ShapeDtypeStruct((BATCH, SEQ), jnp.bool_),
)


def kernel(histories, table0, table1):
    idx = histories.transpose(1, 0, 2).reshape(2, NW, NCH, CHUNK)
    out = _sc_gather(table0, table1, idx[0], idx[1])
    embs = out.reshape(BATCH, SEQ, 2 * EMB)
    mask = _mask_call(histories[:, 0, :])
    return embs, mask


# speedup vs baseline: 1.6986x; 1.0839x over previous
import jax
import jax.numpy as jnp
from jax import lax
from jax.experimental import pallas as pl
from jax.experimental.pallas import tpu as pltpu
from jax.experimental.pallas import tpu_sc as plsc

BATCH = 4096
SEQ = 200
EMB = 32
NTAB = 1000000
NC = 2
NS = 16
NW = NC * NS
ROWS = BATCH * SEQ
RPW = ROWS // NW
CHUNK = 128
NCH = RPW // CHUNK
NBUF = 4


def _sc_gather_body(t0, t1, idx0, idx1, out, idx0_v, idx1_v,
                    b0, b1, b2, b3, g0, g1, g2, g3, w0, w1, w2, w3):
    bufs = (b0, b1, b2, b3)
    gsems = (g0, g1, g2, g3)
    wsems = (w0, w1, w2, w3)
    c = lax.axis_index("c")
    s = lax.axis_index("s")
    wid = s * NC + c
    base = wid * RPW
    pltpu.sync_copy(idx0.at[wid], idx0_v)
    pltpu.sync_copy(idx1.at[wid], idx1_v)

    def field(idx_v, table, col):
        def start_gather(b, j):
            pltpu.async_copy(table.at[idx_v.at[j]], bufs[b], gsems[b])

        def wait_gather(b):
            pltpu.make_async_copy(table.at[idx_v.at[0]], bufs[b], gsems[b]).wait()

        def start_write(b, j):
            pltpu.async_copy(
                bufs[b].at[pl.ds(0, CHUNK), pl.ds(0, EMB)],
                out.at[pl.ds(base + j * CHUNK, CHUNK), pl.ds(col, EMB)],
                wsems[b])

        def wait_write(b):
            pltpu.make_async_copy(
                bufs[b].at[pl.ds(0, CHUNK), pl.ds(0, EMB)],
                out.at[pl.ds(base, CHUNK), pl.ds(col, EMB)],
                wsems[b]).wait()

        for b in range(NBUF):
            start_gather(b, b)

        def outer(jo, carry):
            for b in range(NBUF):
                j = jo * NBUF + b
                wait_gather(b)
                start_write(b, j)
            for b in range(NBUF):
                j = jo * NBUF + b
                wait_write(b)
                start_gather(b, j + NBUF)
            return carry

        lax.fori_loop(0, NCH // NBUF - 1, outer, 0)
        for b in range(NBUF):
            j = NCH - NBUF + b
            wait_gather(b)
            start_write(b, j)
        for b in range(NBUF):
            wait_write(b)

    field(idx0_v, t0, 0)
    field(idx1_v, t1, EMB)


_sc_gather = pl.kernel(
    _sc_gather_body,
    out_type=jax.ShapeDtypeStruct((ROWS, 128), jnp.float32),
    mesh=plsc.VectorSubcoreMesh(core_axis_name="c", subcore_axis_name="s"),
    scratch_types=(
        [pltpu.VMEM((NCH, CHUNK), jnp.int32)] * 2
        + [pltpu.VMEM((CHUNK, 128), jnp.float32)] * NBUF
        + [pltpu.SemaphoreType.DMA] * (2 * NBUF)
    ),
    compiler_params=pltpu.CompilerParams(use_tc_tiling_on_sc=False),
)

_TW_V = 1024


def _twiden_body(x0_ref, x1_ref, o0_ref, o1_ref):
    z = jnp.zeros((_TW_V, 128 - EMB), jnp.float32)
    o0_ref[...] = jnp.concatenate([x0_ref[...].T, z], axis=1)
    o1_ref[...] = jnp.concatenate([x1_ref[...].T, z], axis=1)


_twiden = pl.pallas_call(
    _twiden_body,
    grid=(-(-NTAB // _TW_V),),
    in_specs=[
        pl.BlockSpec((EMB, _TW_V), lambda g: (0, g)),
        pl.BlockSpec((EMB, _TW_V), lambda g: (0, g)),
    ],
    out_specs=[
        pl.BlockSpec((_TW_V, 128), lambda g: (g, 0)),
        pl.BlockSpec((_TW_V, 128), lambda g: (g, 0)),
    ],
    out_shape=[
        jax.ShapeDtypeStruct((NTAB, 128), jnp.float32),
        jax.ShapeDtypeStruct((NTAB, 128), jnp.float32),
    ],
)


def _mask_body(h_ref, m_ref):
    m_ref[...] = h_ref[...] != 0


_mask_call = pl.pallas_call(
    _mask_body,
    out_shape=jax.ShapeDtypeStruct((SEQ, BATCH), jnp.bool_),
)


def kernel(histories, table0, table1):
    t0w, t1w = _twiden(table0.T, table1.T)
    idx = histories.transpose(1, 0, 2).reshape(2, NW, NCH, CHUNK)
    wide = _sc_gather(t0w, t1w, idx[0], idx[1])
    embs = wide.reshape(BATCH, SEQ, 128)[:, :, :2 * EMB]
    ht = histories.transpose(1, 2, 0)
    mask = _mask_call(ht[0]).T
    return embs, mask


# speedup vs baseline: 2.1991x; 1.2946x over previous
import jax
import jax.numpy as jnp
from jax import lax
from jax.experimental import pallas as pl
from jax.experimental.pallas import tpu as pltpu
from jax.experimental.pallas import tpu_sc as plsc

BATCH = 4096
SEQ = 200
EMB = 32
NTAB = 1000000
NC = 2
NS = 16
NW = NC * NS
ROWS = BATCH * SEQ
RPW = ROWS // NW
CHUNK = 128
NCH = RPW // CHUNK
NBUF = 4


def _sc_gather_body(t0, t1, idx0, idx1, out, idx0_v, idx1_v,
                    b0, b1, b2, b3, g0, g1, g2, g3, w0, w1, w2, w3):
    bufs = (b0, b1, b2, b3)
    gsems = (g0, g1, g2, g3)
    wsems = (w0, w1, w2, w3)
    c = lax.axis_index("c")
    s = lax.axis_index("s")
    wid = s * NC + c
    base = wid * RPW
    pltpu.sync_copy(idx0.at[wid], idx0_v)
    pltpu.sync_copy(idx1.at[wid], idx1_v)

    def field(idx_v, table, col):
        def start_gather(b, j):
            pltpu.async_copy(table.at[idx_v.at[j]], bufs[b], gsems[b])

        def wait_gather(b):
            pltpu.make_async_copy(table.at[idx_v.at[0]], bufs[b], gsems[b]).wait()

        def start_write(b, j):
            pltpu.async_copy(
                bufs[b].at[pl.ds(0, CHUNK), pl.ds(0, EMB)],
                out.at[pl.ds(base + j * CHUNK, CHUNK), pl.ds(col, EMB)],
                wsems[b])

        def wait_write(b):
            pltpu.make_async_copy(
                bufs[b].at[pl.ds(0, CHUNK), pl.ds(0, EMB)],
                out.at[pl.ds(base, CHUNK), pl.ds(col, EMB)],
                wsems[b]).wait()

        for b in range(NBUF):
            start_gather(b, b)

        def outer(jo, carry):
            for b in range(NBUF):
                j = jo * NBUF + b
                wait_gather(b)
                start_write(b, j)
            for b in range(NBUF):
                j = jo * NBUF + b
                wait_write(b)
                start_gather(b, j + NBUF)
            return carry

        lax.fori_loop(0, NCH // NBUF - 1, outer, 0)
        for b in range(NBUF):
            j = NCH - NBUF + b
            wait_gather(b)
            start_write(b, j)
        for b in range(NBUF):
            wait_write(b)

    field(idx0_v, t0, 0)
    field(idx1_v, t1, EMB)


_sc_gather = pl.kernel(
    _sc_gather_body,
    out_type=jax.ShapeDtypeStruct((ROWS, 128), jnp.float32),
    mesh=plsc.VectorSubcoreMesh(core_axis_name="c", subcore_axis_name="s"),
    scratch_types=(
        [pltpu.VMEM((NCH, CHUNK), jnp.int32)] * 2
        + [pltpu.VMEM((CHUNK, 128), jnp.float32)] * NBUF
        + [pltpu.SemaphoreType.DMA] * (2 * NBUF)
    ),
    compiler_params=pltpu.CompilerParams(use_tc_tiling_on_sc=False),
)

_TW_V = 4096
_EYE = None


def _twiden_body(x0_ref, x1_ref, o0_ref, o1_ref):
    eye = jax.lax.broadcasted_iota(jnp.int32, (EMB, 128), 0) ==         jax.lax.broadcasted_iota(jnp.int32, (EMB, 128), 1)
    eyef = eye.astype(jnp.float32)
    for x_ref, o_ref in ((x0_ref, o0_ref), (x1_ref, o1_ref)):
        o_ref[...] = jax.lax.dot_general(
            x_ref[...], eyef, (((0,), (0,)), ((), ())),
            preferred_element_type=jnp.float32)


_twiden = pl.pallas_call(
    _twiden_body,
    grid=(-(-NTAB // _TW_V),),
    in_specs=[
        pl.BlockSpec((EMB, _TW_V), lambda g: (0, g)),
        pl.BlockSpec((EMB, _TW_V), lambda g: (0, g)),
    ],
    out_specs=[
        pl.BlockSpec((_TW_V, 128), lambda g: (g, 0)),
        pl.BlockSpec((_TW_V, 128), lambda g: (g, 0)),
    ],
    out_shape=[
        jax.ShapeDtypeStruct((NTAB, 128), jnp.float32),
        jax.ShapeDtypeStruct((NTAB, 128), jnp.float32),
    ],
)


def _mask_body(h_ref, m_ref):
    m_ref[...] = h_ref[...] != 0


_mask_call = pl.pallas_call(
    _mask_body,
    out_shape=jax.ShapeDtypeStruct((SEQ, BATCH), jnp.bool_),
)


def kernel(histories, table0, table1):
    t0w, t1w = _twiden(table0.T, table1.T)
    idx = histories.transpose(1, 0, 2).reshape(2, NW, NCH, CHUNK)
    wide = _sc_gather(t0w, t1w, idx[0], idx[1])
    embs = wide.reshape(BATCH, SEQ, 128)[:, :, :2 * EMB]
    ht = histories.transpose(1, 2, 0)
    mask = _mask_call(ht[0]).T
    return embs, mask
